# baseline (speedup 1.0000x reference)
import jax
import jax.numpy as jnp
from jax import lax
from jax.experimental import pallas as pl
from jax.experimental.pallas import tpu as pltpu
from jax.experimental.pallas import tpu_sc as plsc

B = 16384
S = 2048
T = B * S
LOG2S = 11
LOG2SEG = 20

NC = 2
NT = 16
SEG = 1 << LOG2SEG
NSEG = NSEG_PER_SC = T // (SEG * NC)
SLICE_W = SEG // NT
DUMP = SEG

P = 62_720
CH = 2240
NCH = P // CH
INNER = CH // 16
NP = P * NT

CAP = 2560
HCAP = CAP // 2


def _hist_body(pairs_hbm, zeros_hbm, dump_hbm, out_hbm,
               bk_v, pstage_v, val_v, zval_v, cnt_v, cntb_v,
               acc_sh, sem):
    c = lax.axis_index("c")
    s = lax.axis_index("s")
    iota = lax.iota(jnp.int32, 16)

    pltpu.sync_copy(dump_hbm, bk_v)
    cnt_v[...] = jnp.zeros((16,), jnp.int32)
    cntb_v[...] = jnp.zeros((16,), jnp.int32)
    def vfill(k, _):
        val_v[pl.ds(k * 16, 16)] = jnp.full((16,), 1.0, jnp.float32)
        zval_v[pl.ds(k * 16, 16)] = jnp.zeros((16,), jnp.float32)
        return 0

    lax.fori_loop(0, CAP // 16, vfill, 0)
    my_w = pl.multiple_of(s * SLICE_W, 8)
    pltpu.sync_copy(zeros_hbm.at[s], acc_sh.at[pl.ds(my_w, SLICE_W)])

    def chunk_body(m, _):
        pltpu.sync_copy(pairs_hbm.at[s, m], pstage_v)

        def bin16(k, cref, half_base):
            pbase = k * 32 + iota * 2
            r16 = plsc.load_gather(pstage_v, [pbase])
            c16 = plsc.load_gather(pstage_v, [pbase + 1])
            flat = jnp.bitwise_or(lax.shift_left(r16, LOG2S), c16)
            gseg = lax.shift_right_logical(flat, LOG2SEG)
            lseg = gseg - c * NSEG
            valid = (lseg >= 0) & (lseg < NSEG)
            key = jnp.where(valid, lseg, NSEG)
            loc = jnp.bitwise_and(flat, SEG - 1)
            occ, lastm = plsc.scan_count(key, mask=valid)
            kg = jnp.minimum(key, NSEG - 1)
            cnt16 = plsc.load_gather(cref, [kg])
            pos = cnt16 + occ - 1
            ok = valid & (pos < HCAP)
            plsc.store_scatter(bk_v, [kg * CAP + half_base + pos], loc,
                               mask=ok)
            plsc.store_scatter(cref, [kg], jnp.minimum(pos + 1, HCAP),
                               mask=lastm & valid)
            return 0

        def bin32(k, _):
            bin16(2 * k, cnt_v, 0)
            bin16(2 * k + 1, cntb_v, HCAP)
            return 0

        lax.fori_loop(0, INNER // 2, bin32, 0)
        return 0

    lax.fori_loop(0, NCH, chunk_body, 0)
    plsc.subcore_barrier()

    def seg_body(l, _):
        gs = c * NSEG + l

        lo = pl.multiple_of(l * CAP, 8)
        pltpu.sync_copy(val_v, acc_sh.at[bk_v.at[pl.ds(lo, CAP)]], add=True)
        plsc.subcore_barrier()

        r0 = (gs * NT + s) * 32

        def wrow(j, _):
            pltpu.async_copy(
                acc_sh.at[pl.ds(pl.multiple_of(my_w + j * S, 8), S)],
                out_hbm.at[r0 + j], sem)
            return 0

        def wdrain(j, _):
            pltpu.make_async_copy(
                acc_sh.at[pl.ds(pl.multiple_of(my_w + j * S, 8), S)],
                out_hbm.at[r0 + j], sem).wait()
            return 0

        lax.fori_loop(0, 32, wrow, 0)
        lax.fori_loop(0, 32, wdrain, 0)

        pltpu.sync_copy(zeros_hbm.at[s], acc_sh.at[pl.ds(my_w, SLICE_W)])
        plsc.subcore_barrier()
        return 0

    lax.fori_loop(0, NSEG, seg_body, 0)


@jax.jit
def _hist(pairs, zeros_seg, dumpfill):
    mesh = plsc.VectorSubcoreMesh(core_axis_name="c", subcore_axis_name="s")
    return pl.kernel(
        _hist_body,
        out_type=jax.ShapeDtypeStruct((B, S), jnp.float32),
        mesh=mesh,
        compiler_params=pltpu.CompilerParams(needs_layout_passes=False),
        scratch_types=[
            pltpu.VMEM((NSEG * CAP,), jnp.int32),
            pltpu.VMEM((2 * CH,), jnp.int32),
            pltpu.VMEM((CAP,), jnp.float32),
            pltpu.VMEM((CAP,), jnp.float32),
            pltpu.VMEM((16,), jnp.int32),
            pltpu.VMEM((16,), jnp.int32),
            pltpu.VMEM_SHARED((SEG + 128,), jnp.float32),
            pltpu.SemaphoreType.DMA,
        ],
    )(pairs, zeros_seg, dumpfill)


def kernel(index, dest_shape):
    del dest_shape
    pad = NP - index.shape[0]
    pairs = jnp.pad(index.astype(jnp.int32), ((0, pad), (0, 0)),
                    constant_values=B).reshape(NT, NCH, 2 * CH)
    zeros_seg = jnp.zeros((NT, SLICE_W), jnp.float32)
    dumpfill = DUMP + (jnp.arange(NSEG * CAP, dtype=jnp.int32) % 128)
    out = _hist(pairs, zeros_seg, dumpfill)
    return out[..., None]


# speedup vs baseline: 4.9373x; 4.9373x over previous
import jax
import jax.numpy as jnp
from jax import lax
from jax.experimental import pallas as pl
from jax.experimental.pallas import tpu as pltpu
from jax.experimental.pallas import tpu_sc as plsc

B = 16384
S = 2048
T = B * S
LOG2S = 11
LOG2SEG = 20

NC = 2
NT = 16
SEG = 1 << LOG2SEG
NSEG = NSEG_PER_SC = T // (SEG * NC)
SLICE_W = SEG // NT
DUMP = SEG

P = 62_720
CH = 2240
NCH = P // CH
INNER = CH // 16
NP = P * NT

CAP = 2560
HCAP = CAP // 2


def _hist_body(rows_hbm, cols_hbm, zeros_hbm, dump_hbm, out_hbm,
               bk_v, rstage_v, cstage_v, val_v, zval_v, cnt_v, cntb_v,
               acc_sh, sem):
    c = lax.axis_index("c")
    s = lax.axis_index("s")

    pltpu.sync_copy(dump_hbm, bk_v)
    cnt_v[...] = jnp.zeros((16,), jnp.int32)
    cntb_v[...] = jnp.zeros((16,), jnp.int32)
    def vfill(k, _):
        val_v[pl.ds(k * 16, 16)] = jnp.full((16,), 1.0, jnp.float32)
        zval_v[pl.ds(k * 16, 16)] = jnp.zeros((16,), jnp.float32)
        return 0

    lax.fori_loop(0, CAP // 16, vfill, 0)
    my_w = pl.multiple_of(s * SLICE_W, 8)
    pltpu.sync_copy(zeros_hbm.at[s], acc_sh.at[pl.ds(my_w, SLICE_W)])

    def chunk_body(m, _):
        pltpu.sync_copy(rows_hbm.at[s, m], rstage_v)
        pltpu.sync_copy(cols_hbm.at[s, m], cstage_v)

        def bin16(k, cref, half_base):
            r16 = rstage_v[pl.ds(k * 16, 16)]
            c16 = cstage_v[pl.ds(k * 16, 16)]
            flat = jnp.bitwise_or(lax.shift_left(r16, LOG2S), c16)
            gseg = lax.shift_right_logical(flat, LOG2SEG)
            lseg = gseg - c * NSEG
            valid = (lseg >= 0) & (lseg < NSEG)
            key = jnp.where(valid, lseg, NSEG)
            loc = jnp.bitwise_and(flat, SEG - 1)
            occ, lastm = plsc.scan_count(key, mask=valid)
            kg = jnp.minimum(key, NSEG - 1)
            cnt16 = plsc.load_gather(cref, [kg])
            pos = cnt16 + occ - 1
            ok = valid & (pos < HCAP)
            plsc.store_scatter(bk_v, [kg * CAP + half_base + pos], loc,
                               mask=ok)
            plsc.store_scatter(cref, [kg], jnp.minimum(pos + 1, HCAP),
                               mask=lastm & valid)
            return 0

        def bin32(k, _):
            bin16(2 * k, cnt_v, 0)
            bin16(2 * k + 1, cntb_v, HCAP)
            return 0

        lax.fori_loop(0, INNER // 2, bin32, 0)
        return 0

    lax.fori_loop(0, NCH, chunk_body, 0)
    plsc.subcore_barrier()

    def seg_body(l, _):
        gs = c * NSEG + l

        lo = pl.multiple_of(l * CAP, 8)
        pltpu.sync_copy(val_v, acc_sh.at[bk_v.at[pl.ds(lo, CAP)]], add=True)
        plsc.subcore_barrier()

        r0 = (gs * NT + s) * 32

        def wrow(j, _):
            pltpu.async_copy(
                acc_sh.at[pl.ds(pl.multiple_of(my_w + j * S, 8), S)],
                out_hbm.at[r0 + j], sem)
            return 0

        def wdrain(j, _):
            pltpu.make_async_copy(
                acc_sh.at[pl.ds(pl.multiple_of(my_w + j * S, 8), S)],
                out_hbm.at[r0 + j], sem).wait()
            return 0

        lax.fori_loop(0, 32, wrow, 0)
        lax.fori_loop(0, 32, wdrain, 0)

        pltpu.sync_copy(zeros_hbm.at[s], acc_sh.at[pl.ds(my_w, SLICE_W)])
        plsc.subcore_barrier()
        return 0

    lax.fori_loop(0, NSEG, seg_body, 0)


@jax.jit
def _hist(rows, cols, zeros_seg, dumpfill):
    mesh = plsc.VectorSubcoreMesh(core_axis_name="c", subcore_axis_name="s")
    return pl.kernel(
        _hist_body,
        out_type=jax.ShapeDtypeStruct((B, S), jnp.float32),
        mesh=mesh,
        compiler_params=pltpu.CompilerParams(needs_layout_passes=False),
        scratch_types=[
            pltpu.VMEM((NSEG * CAP,), jnp.int32),
            pltpu.VMEM((CH,), jnp.int32),
            pltpu.VMEM((CH,), jnp.int32),
            pltpu.VMEM((CAP,), jnp.float32),
            pltpu.VMEM((CAP,), jnp.float32),
            pltpu.VMEM((16,), jnp.int32),
            pltpu.VMEM((16,), jnp.int32),
            pltpu.VMEM_SHARED((SEG + 128,), jnp.float32),
            pltpu.SemaphoreType.DMA,
        ],
    )(rows, cols, zeros_seg, dumpfill)


def kernel(index, dest_shape):
    del dest_shape
    pad = NP - index.shape[0]
    ixt = jnp.pad(index.astype(jnp.int32).T, ((0, 0), (0, pad)),
                  constant_values=B)
    rows, cols = ixt[0], ixt[1]
    rows = rows.reshape(NT, NCH, CH)
    cols = cols.reshape(NT, NCH, CH)
    zeros_seg = jnp.zeros((NT, SLICE_W), jnp.float32)
    dumpfill = DUMP + (jnp.arange(NSEG * CAP, dtype=jnp.int32) % 128)
    out = _hist(rows, cols, zeros_seg, dumpfill)
    return out[..., None]
